# baseline (speedup 1.0000x reference)
import functools

import jax
import jax.numpy as jnp
from jax import lax
from jax.experimental import pallas as pl
from jax.experimental.pallas import tpu as pltpu
from jax.experimental.pallas import tpu_sc as plsc

H = 64
NIN = 26
NCTX = 20
RPI = 48
C = 4
ROWS = RPI * C
NBUF = 4
PSW = 32
L = 16
NQ = H // L


@functools.lru_cache(maxsize=None)
def _make_sc_scores(B: int, OFF: int):
    info = plsc.get_sparse_core_info()
    NC, NS = info.num_cores, info.num_subcores
    NW = NC * NS
    assert B % (NW * C) == 0
    BPW = B // NW
    NCH = BPW // C

    mesh = plsc.VectorSubcoreMesh(core_axis_name="c", subcore_axis_name="s")

    @functools.partial(
        pl.kernel,
        mesh=mesh,
        compiler_params=pltpu.CompilerParams(
            needs_layout_passes=False, use_tc_tiling_on_sc=False),
        out_type=jax.ShapeDtypeStruct((B, PSW), jnp.float32),
        scratch_types=[
            pltpu.VMEM((NCH, ROWS), jnp.int32),
            pltpu.VMEM((NBUF, ROWS, H), jnp.bfloat16),
            pltpu.VMEM((BPW, PSW), jnp.float32),
            pltpu.SemaphoreType.DMA,
            pltpu.SemaphoreType.DMA,
            pltpu.SemaphoreType.DMA,
            pltpu.SemaphoreType.DMA,
        ],
    )
    def sc_scores(idx_hbm, table_hbm, ps_hbm, idx_v, rows_v, ps_v,
                  s0, s1, s2, s3):
        sems = [s0, s1, s2, s3]
        wid = lax.axis_index("s") * NC + lax.axis_index("c")
        pltpu.sync_copy(idx_hbm.at[pl.ds(wid * NCH, NCH)], idx_v)

        def gather_start(j, ch):
            def issue(t, _):
                iv = idx_v[ch, pl.ds(t * L, L)]
                for e in range(L):
                    s = iv[e]
                    pltpu.make_async_copy(
                        table_hbm.at[pl.ds(s >> 1, 1),
                                     pl.ds((s & 1) * H, H)],
                        rows_v.at[j, pl.ds(t * L + e, 1), :],
                        sems[j]).start()
                return 0

            lax.fori_loop(0, ROWS // L, issue, 0)

        def gather_wait(j):
            pltpu.make_async_copy(
                table_hbm.at[pl.ds(0, ROWS), pl.ds(0, H)],
                rows_v.at[j], sems[j]).wait()

        for j in range(NBUF):
            gather_start(j, j)

        lane_iota = lax.iota(jnp.int32, 16)

        def row_f32(jd, r):
            u = plsc.unpack(rows_v[jd, r, pl.ds(0, 32)],
                            format=plsc.PackFormat.INTERLEAVED,
                            preferred_element_type=jnp.float32)
            v = plsc.unpack(rows_v[jd, r, pl.ds(32, 32)],
                            format=plsc.PackFormat.INTERLEAVED,
                            preferred_element_type=jnp.float32)
            return [u[0], u[1], v[0], v[1]]

        def process(jd, ch):
            for k in range(C):
                base = k * RPI
                iv = [idx_v[ch, pl.ds(base + 16 * t, 16)]
                      for t in range(RPI // L)]
                mv = [jnp.where(v > 0, 1.0, 0.0) for v in iv]

                ctx = [jnp.zeros((L,), jnp.float32)] * NQ
                for w in range(NCTX):
                    rr = NIN + w
                    m = mv[rr // 16][rr % 16]
                    hv = row_f32(jd, base + rr)
                    for q in range(NQ):
                        ctx[q] = ctx[q] + hv[q] * m
                ctx = [cq * (1.0 / NCTX) for cq in ctx]

                ps0 = jnp.zeros((L,), jnp.float32)
                ps1 = jnp.zeros((L,), jnp.float32)
                for nn in range(NIN):
                    hv = row_f32(jd, base + nn)
                    t = hv[0] * ctx[0]
                    for q in range(1, NQ):
                        t = t + hv[q] * ctx[q]
                    p = jnp.sum(t) * mv[nn // 16][nn % 16]
                    if nn < 16:
                        ps0 = jnp.where(lane_iota == nn, p, ps0)
                    else:
                        ps1 = jnp.where(lane_iota == (nn - 16), p, ps1)
                il = ch * C + k
                ps_v[il, pl.ds(0, L)] = ps0
                ps_v[il, pl.ds(L, L)] = ps1

        def outer(ch, _):
            jd = lax.rem(ch, NBUF)
            for j in range(NBUF):
                @pl.when(jd == j)
                def _():
                    gather_wait(j)
            process(jd, ch)

            @pl.when(ch + NBUF < NCH)
            def _():
                for j in range(NBUF):
                    @pl.when(jd == j)
                    def _():
                        gather_start(j, ch + NBUF)
            return 0

        lax.fori_loop(0, NCH, outer, 0)
        pltpu.sync_copy(ps_v, ps_hbm.at[pl.ds(wid * BPW, BPW)])

    return sc_scores


_PB = 1024


def _repack_body(a_ref, b_ref, out_ref):
    a = jnp.swapaxes(a_ref[...], 0, 1)
    b = jnp.swapaxes(b_ref[...], 0, 1)
    out_ref[...] = jnp.concatenate([a, b], axis=1).astype(jnp.bfloat16)


def _pair_off(V):
    return ((V // 2 + _PB - 1) // _PB) * _PB


def _repack(wordemb_t):
    V = wordemb_t.shape[1]
    off = _pair_off(V)
    grid = off // _PB
    return pl.pallas_call(
        _repack_body,
        grid=(grid,),
        in_specs=[
            pl.BlockSpec((H, _PB), lambda g: (0, g)),
            pl.BlockSpec((H, _PB), lambda g: (0, g + grid)),
        ],
        out_specs=pl.BlockSpec((_PB, 2 * H), lambda g: (g, 0)),
        out_shape=jax.ShapeDtypeStruct((off, 2 * H), jnp.bfloat16),
    )(wordemb_t, wordemb_t)


def _loss_body(ps_ref, out_ref):
    x = ps_ref[...]
    col = lax.broadcasted_iota(jnp.int32, x.shape, 1)
    xm = jnp.where(col < NIN, x, -1e30)
    m = jnp.max(xm, axis=1, keepdims=True)
    se = jnp.sum(jnp.exp(xm - m), axis=1, keepdims=True)
    lse = m + jnp.log(se)
    out_ref[...] = jnp.mean(lse - x[:, 0:1]).reshape(1, 1)


def kernel(targets, contexts, negtives, wordemb):
    B = targets.shape[0]
    V = wordemb.shape[0]
    idx_all = jnp.concatenate(
        [
            targets.astype(jnp.int32).reshape(B, 1),
            negtives.astype(jnp.int32).reshape(B, -1),
            contexts.astype(jnp.int32).reshape(B, -1),
            jnp.zeros((B, RPI - NIN - NCTX), jnp.int32),
        ],
        axis=1,
    ).reshape(B // C, ROWS)
    table_bf = wordemb.astype(jnp.bfloat16).reshape(V // 2, 2 * H)
    ps = _make_sc_scores(B, V)(idx_all, table_bf)
    loss = pl.pallas_call(
        _loss_body,
        out_shape=jax.ShapeDtypeStruct((1, 1), jnp.float32),
    )(ps)
    return loss[0, 0]


# speedup vs baseline: 1.0017x; 1.0017x over previous
import functools

import jax
import jax.numpy as jnp
from jax import lax
from jax.experimental import pallas as pl
from jax.experimental.pallas import tpu as pltpu
from jax.experimental.pallas import tpu_sc as plsc

H = 64
NIN = 26
NCTX = 20
RPI = 48
C = 4
ROWS = RPI * C
NBUF = 4
PSW = 32
L = 16
NQ = H // L


@functools.lru_cache(maxsize=None)
def _make_sc_scores(B: int, V: int):
    info = plsc.get_sparse_core_info()
    NC, NS = info.num_cores, info.num_subcores
    NW = NC * NS
    assert B % (NW * C) == 0
    BPW = B // NW
    NCH = BPW // C

    mesh = plsc.VectorSubcoreMesh(core_axis_name="c", subcore_axis_name="s")

    @functools.partial(
        pl.kernel,
        mesh=mesh,
        compiler_params=pltpu.CompilerParams(
            needs_layout_passes=False, use_tc_tiling_on_sc=False),
        out_type=jax.ShapeDtypeStruct((B, PSW), jnp.float32),
        scratch_types=[
            pltpu.VMEM((NCH, ROWS), jnp.int32),
            pltpu.VMEM((NBUF, ROWS, H), jnp.bfloat16),
            pltpu.VMEM((BPW, PSW), jnp.float32),
            pltpu.SemaphoreType.DMA,
            pltpu.SemaphoreType.DMA,
            pltpu.SemaphoreType.DMA,
            pltpu.SemaphoreType.DMA,
        ],
    )
    def sc_scores(idx_hbm, table_hbm, ps_hbm, idx_v, rows_v, ps_v,
                  s0, s1, s2, s3):
        sems = [s0, s1, s2, s3]
        wid = lax.axis_index("s") * NC + lax.axis_index("c")
        pltpu.sync_copy(idx_hbm.at[pl.ds(wid * NCH, NCH)], idx_v)

        def gather(j, ch):
            return pltpu.make_async_copy(
                table_hbm.at[idx_v.at[ch]], rows_v.at[j], sems[j])

        for j in range(NBUF):
            gather(j, j).start()

        lane_iota = lax.iota(jnp.int32, 16)

        def row_f32(jd, r):
            u = plsc.unpack(rows_v[jd, r, pl.ds(0, 32)],
                            format=plsc.PackFormat.INTERLEAVED,
                            preferred_element_type=jnp.float32)
            v = plsc.unpack(rows_v[jd, r, pl.ds(32, 32)],
                            format=plsc.PackFormat.INTERLEAVED,
                            preferred_element_type=jnp.float32)
            return [u[0], u[1], v[0], v[1]]

        def process(jd, ch):
            for k in range(C):
                base = k * RPI
                iv = [idx_v[ch, pl.ds(base + 16 * t, 16)]
                      for t in range(RPI // L)]
                mv = [jnp.where(v > 0, 1.0, 0.0) for v in iv]

                ctx = [jnp.zeros((L,), jnp.float32)] * NQ
                for w in range(NCTX):
                    rr = NIN + w
                    m = mv[rr // 16][rr % 16]
                    hv = row_f32(jd, base + rr)
                    for q in range(NQ):
                        ctx[q] = ctx[q] + hv[q] * m
                ctx = [cq * (1.0 / NCTX) for cq in ctx]

                ps0 = jnp.zeros((L,), jnp.float32)
                ps1 = jnp.zeros((L,), jnp.float32)
                for nn in range(NIN):
                    hv = row_f32(jd, base + nn)
                    t = hv[0] * ctx[0]
                    for q in range(1, NQ):
                        t = t + hv[q] * ctx[q]
                    p = jnp.sum(t) * mv[nn // 16][nn % 16]
                    if nn < 16:
                        ps0 = jnp.where(lane_iota == nn, p, ps0)
                    else:
                        ps1 = jnp.where(lane_iota == (nn - 16), p, ps1)
                il = ch * C + k
                ps_v[il, pl.ds(0, L)] = ps0
                ps_v[il, pl.ds(L, L)] = ps1

        def outer(ch, _):
            jd = lax.rem(ch, NBUF)
            for j in range(NBUF):
                @pl.when(jd == j)
                def _():
                    gather(j, ch).wait()
            process(jd, ch)

            @pl.when(ch + NBUF < NCH)
            def _():
                for j in range(NBUF):
                    @pl.when(jd == j)
                    def _():
                        gather(j, ch + NBUF).start()
            return 0

        lax.fori_loop(0, NCH, outer, 0)
        pltpu.sync_copy(ps_v, ps_hbm.at[pl.ds(wid * BPW, BPW)])

    return sc_scores


def _loss_body(ps_ref, out_ref):
    x = ps_ref[...]
    col = lax.broadcasted_iota(jnp.int32, x.shape, 1)
    xm = jnp.where(col < NIN, x, -1e30)
    m = jnp.max(xm, axis=1, keepdims=True)
    se = jnp.sum(jnp.exp(xm - m), axis=1, keepdims=True)
    lse = m + jnp.log(se)
    out_ref[...] = jnp.mean(lse - x[:, 0:1]).reshape(1, 1)


def kernel(targets, contexts, negtives, wordemb):
    B = targets.shape[0]
    V = wordemb.shape[0]
    idx_all = jnp.concatenate(
        [
            targets.astype(jnp.int32).reshape(B, 1),
            negtives.astype(jnp.int32).reshape(B, -1),
            contexts.astype(jnp.int32).reshape(B, -1),
            jnp.zeros((B, RPI - NIN - NCTX), jnp.int32),
        ],
        axis=1,
    ).reshape(B // C, ROWS)
    table_bf = wordemb.astype(jnp.bfloat16)
    ps = _make_sc_scores(B, V)(idx_all, table_bf)
    loss = pl.pallas_call(
        _loss_body,
        out_shape=jax.ShapeDtypeStruct((1, 1), jnp.float32),
    )(ps)
    return loss[0, 0]
